# baseline (speedup 1.0000x reference)
import functools

import jax
import jax.numpy as jnp
from jax import lax
from jax.experimental import pallas as pl
from jax.experimental.pallas import tpu as pltpu
from jax.experimental.pallas import tpu_sc as plsc

N, C, H, W = 16, 512, 32, 32
NC_SC = 2
NS_SC = 16
NW = NC_SC * NS_SC
RW = (N * H) // NW
L = 16


def _mlp_body(mask_ref, w1_ref, b1_ref, w2_ref, b2_ref, wfc_ref, bfc_ref,
              m_ref, s_ref):
    h = jax.nn.relu(
        jnp.dot(mask_ref[...], w1_ref[...],
                preferred_element_type=jnp.float32) + b1_ref[...])
    m = jax.nn.sigmoid(
        jnp.dot(h, w2_ref[...],
                preferred_element_type=jnp.float32) + b2_ref[...])
    s = jax.nn.sigmoid(
        jnp.dot(m, wfc_ref[...],
                preferred_element_type=jnp.float32) + bfc_ref[...])
    m_ref[...] = m
    s_ref[...] = s


def _sc_body(sel_hbm, lst_hbm, gui_hbm, out_l_hbm, out_g_hbm,
             lbuf, gbuf, selv, lin, gin, lout, gout):
    wid = lax.axis_index("s") * NC_SC + lax.axis_index("c")
    row0 = wid * RW
    n = wid // (H // RW)

    def in_cp(i, slot):
        return (
            pltpu.make_async_copy(lst_hbm.at[pl.ds(row0 + i, 1)],
                                  lbuf.at[slot], lin.at[slot]),
            pltpu.make_async_copy(gui_hbm.at[pl.ds(row0 + i, 1)],
                                  gbuf.at[slot], gin.at[slot]),
        )

    def out_cp(i, slot):
        return (
            pltpu.make_async_copy(lbuf.at[slot],
                                  out_l_hbm.at[pl.ds(row0 + i, 1)],
                                  lout.at[slot]),
            pltpu.make_async_copy(gbuf.at[slot],
                                  out_g_hbm.at[pl.ds(row0 + i, 1)],
                                  gout.at[slot]),
        )

    a, b = in_cp(0, 0)
    a.start()
    b.start()
    pltpu.sync_copy(sel_hbm.at[pl.ds(n, 1)], selv)

    def compute(slot):
        def jbody(j, _):
            msk = selv[0, pl.ds(j * L, L)] > 0.5

            def wbody(w, _):
                l = lbuf[slot, 0, w, pl.ds(j * L, L)]
                g = gbuf[slot, 0, w, pl.ds(j * L, L)]
                lbuf[slot, 0, w, pl.ds(j * L, L)] = jnp.where(msk, g, l)
                gbuf[slot, 0, w, pl.ds(j * L, L)] = jnp.where(msk, l, g)
                return 0

            return lax.fori_loop(0, W, wbody, 0)

        lax.fori_loop(0, C // L, jbody, 0)

    for i in range(RW):
        slot = i % 2
        a, b = in_cp(i, slot)
        a.wait()
        b.wait()
        compute(slot)
        a, b = out_cp(i, slot)
        a.start()
        b.start()
        if i + 1 < RW:
            s2 = (i + 1) % 2
            if i + 1 >= 2:
                a, b = out_cp(i - 1, s2)
                a.wait()
                b.wait()
            a, b = in_cp(i + 1, s2)
            a.start()
            b.start()

    a, b = out_cp(RW - 2, (RW - 2) % 2)
    a.wait()
    b.wait()
    a, b = out_cp(RW - 1, (RW - 1) % 2)
    a.wait()
    b.wait()


def kernel(lst, gui, mask, W1, b1, W2, b2, Wfc, bfc):
    m, s = pl.pallas_call(
        _mlp_body,
        out_shape=(
            jax.ShapeDtypeStruct((N, C), jnp.float32),
            jax.ShapeDtypeStruct((N, C), jnp.float32),
        ),
    )(mask, W1, b1.reshape(1, C), W2, b2.reshape(1, C),
      Wfc, bfc.reshape(1, C))

    lst3 = lst.transpose(0, 2, 3, 1).reshape(N * H, W, C)
    gui3 = gui.transpose(0, 2, 3, 1).reshape(N * H, W, C)

    sc_exchange = pl.kernel(
        _sc_body,
        out_type=(
            jax.ShapeDtypeStruct((N * H, W, C), jnp.float32),
            jax.ShapeDtypeStruct((N * H, W, C), jnp.float32),
        ),
        mesh=plsc.VectorSubcoreMesh(core_axis_name="c", subcore_axis_name="s"),
        scratch_types=[
            pltpu.VMEM((2, 1, W, C), jnp.float32),
            pltpu.VMEM((2, 1, W, C), jnp.float32),
            pltpu.VMEM((1, C), jnp.float32),
            pltpu.SemaphoreType.DMA((2,)),
            pltpu.SemaphoreType.DMA((2,)),
            pltpu.SemaphoreType.DMA((2,)),
            pltpu.SemaphoreType.DMA((2,)),
        ],
    )
    out_l3, out_g3 = sc_exchange(s, lst3, gui3)

    out_lst = out_l3.reshape(N, H, W, C).transpose(0, 3, 1, 2)
    out_gui = out_g3.reshape(N, H, W, C).transpose(0, 3, 1, 2)
    return (out_lst, out_gui, m)


# speedup vs baseline: 1.9446x; 1.9446x over previous
import functools

import jax
import jax.numpy as jnp
from jax import lax
from jax.experimental import pallas as pl
from jax.experimental.pallas import tpu as pltpu
from jax.experimental.pallas import tpu_sc as plsc

N, C, H, W = 16, 512, 32, 32
NC_SC = 2
NS_SC = 16
NW = NC_SC * NS_SC
RW = (N * H) // NW
L = 16


def _mlp_body(mask_ref, w1_ref, b1_ref, w2_ref, b2_ref, wfc_ref, bfc_ref,
              m_ref, s_ref):
    h = jax.nn.relu(
        jnp.dot(mask_ref[...], w1_ref[...],
                preferred_element_type=jnp.float32) + b1_ref[...])
    m = jax.nn.sigmoid(
        jnp.dot(h, w2_ref[...],
                preferred_element_type=jnp.float32) + b2_ref[...])
    s = jax.nn.sigmoid(
        jnp.dot(m, wfc_ref[...],
                preferred_element_type=jnp.float32) + bfc_ref[...])
    m_ref[...] = m
    s_ref[...] = s


def _sc_body(sel_hbm, lst_hbm, gui_hbm, out_l_hbm, out_g_hbm,
             lbuf, gbuf, selv, lin, gin, lout, gout):
    wid = lax.axis_index("s") * NC_SC + lax.axis_index("c")
    row0 = wid * RW
    n = wid // (H // RW)

    def in_cp(i, slot):
        return (
            pltpu.make_async_copy(lst_hbm.at[pl.ds(row0 + i, 1)],
                                  lbuf.at[slot], lin.at[slot]),
            pltpu.make_async_copy(gui_hbm.at[pl.ds(row0 + i, 1)],
                                  gbuf.at[slot], gin.at[slot]),
        )

    def out_cp(i, slot):
        return (
            pltpu.make_async_copy(lbuf.at[slot],
                                  out_l_hbm.at[pl.ds(row0 + i, 1)],
                                  lout.at[slot]),
            pltpu.make_async_copy(gbuf.at[slot],
                                  out_g_hbm.at[pl.ds(row0 + i, 1)],
                                  gout.at[slot]),
        )

    a, b = in_cp(0, 0)
    a.start()
    b.start()
    pltpu.sync_copy(sel_hbm.at[pl.ds(n, 1)], selv)

    def compute(slot):
        @plsc.parallel_loop(0, C // L, unroll=2)
        def _jloop(j):
            msk = selv[0, pl.ds(j * L, L)] > 0.5

            @plsc.parallel_loop(0, W, unroll=8)
            def _wloop(w):
                l = lbuf[slot, 0, w, pl.ds(j * L, L)]
                g = gbuf[slot, 0, w, pl.ds(j * L, L)]
                lbuf[slot, 0, w, pl.ds(j * L, L)] = jnp.where(msk, g, l)
                gbuf[slot, 0, w, pl.ds(j * L, L)] = jnp.where(msk, l, g)

    for i in range(RW):
        slot = i % 2
        a, b = in_cp(i, slot)
        a.wait()
        b.wait()
        compute(slot)
        a, b = out_cp(i, slot)
        a.start()
        b.start()
        if i + 1 < RW:
            s2 = (i + 1) % 2
            if i + 1 >= 2:
                a, b = out_cp(i - 1, s2)
                a.wait()
                b.wait()
            a, b = in_cp(i + 1, s2)
            a.start()
            b.start()

    a, b = out_cp(RW - 2, (RW - 2) % 2)
    a.wait()
    b.wait()
    a, b = out_cp(RW - 1, (RW - 1) % 2)
    a.wait()
    b.wait()


def kernel(lst, gui, mask, W1, b1, W2, b2, Wfc, bfc):
    m, s = pl.pallas_call(
        _mlp_body,
        out_shape=(
            jax.ShapeDtypeStruct((N, C), jnp.float32),
            jax.ShapeDtypeStruct((N, C), jnp.float32),
        ),
    )(mask, W1, b1.reshape(1, C), W2, b2.reshape(1, C),
      Wfc, bfc.reshape(1, C))

    lst3 = lst.transpose(0, 2, 3, 1).reshape(N * H, W, C)
    gui3 = gui.transpose(0, 2, 3, 1).reshape(N * H, W, C)

    sc_exchange = pl.kernel(
        _sc_body,
        out_type=(
            jax.ShapeDtypeStruct((N * H, W, C), jnp.float32),
            jax.ShapeDtypeStruct((N * H, W, C), jnp.float32),
        ),
        mesh=plsc.VectorSubcoreMesh(core_axis_name="c", subcore_axis_name="s"),
        scratch_types=[
            pltpu.VMEM((2, 1, W, C), jnp.float32),
            pltpu.VMEM((2, 1, W, C), jnp.float32),
            pltpu.VMEM((1, C), jnp.float32),
            pltpu.SemaphoreType.DMA((2,)),
            pltpu.SemaphoreType.DMA((2,)),
            pltpu.SemaphoreType.DMA((2,)),
            pltpu.SemaphoreType.DMA((2,)),
        ],
    )
    out_l3, out_g3 = sc_exchange(s, lst3, gui3)

    out_lst = out_l3.reshape(N, H, W, C).transpose(0, 3, 1, 2)
    out_gui = out_g3.reshape(N, H, W, C).transpose(0, 3, 1, 2)
    return (out_lst, out_gui, m)


# speedup vs baseline: 2.1839x; 1.1230x over previous
import functools

import jax
import jax.numpy as jnp
from jax import lax
from jax.experimental import pallas as pl
from jax.experimental.pallas import tpu as pltpu
from jax.experimental.pallas import tpu_sc as plsc

N, C, H, W = 16, 512, 32, 32
NC_SC = 2
NS_SC = 16
NW = NC_SC * NS_SC
RW = (N * H) // NW
L = 16
SC_S = 3
SC_PF = 2


def _mlp_body(mask_ref, w1_ref, b1_ref, w2_ref, b2_ref, wfc_ref, bfc_ref,
              m_ref, s_ref):
    h = jax.nn.relu(
        jnp.dot(mask_ref[...], w1_ref[...],
                preferred_element_type=jnp.float32) + b1_ref[...])
    m = jax.nn.sigmoid(
        jnp.dot(h, w2_ref[...],
                preferred_element_type=jnp.float32) + b2_ref[...])
    s = jax.nn.sigmoid(
        jnp.dot(m, wfc_ref[...],
                preferred_element_type=jnp.float32) + bfc_ref[...])
    m_ref[...] = m
    s_ref[...] = s


def _sc_body(sel_hbm, lst_hbm, gui_hbm, out_l_hbm, out_g_hbm,
             lbuf, gbuf, selv, lin, gin, lout, gout):
    wid = lax.axis_index("s") * NC_SC + lax.axis_index("c")
    row0 = wid * RW
    n = wid // (H // RW)

    def in_cp(i, slot):
        return (
            pltpu.make_async_copy(lst_hbm.at[pl.ds(row0 + i, 1)],
                                  lbuf.at[slot], lin.at[slot]),
            pltpu.make_async_copy(gui_hbm.at[pl.ds(row0 + i, 1)],
                                  gbuf.at[slot], gin.at[slot]),
        )

    def out_cp(i, slot):
        return (
            pltpu.make_async_copy(lbuf.at[slot],
                                  out_l_hbm.at[pl.ds(row0 + i, 1)],
                                  lout.at[slot]),
            pltpu.make_async_copy(gbuf.at[slot],
                                  out_g_hbm.at[pl.ds(row0 + i, 1)],
                                  gout.at[slot]),
        )

    for k in range(SC_PF):
        a, b = in_cp(k, k)
        a.start()
        b.start()
    pltpu.sync_copy(sel_hbm.at[pl.ds(n, 1)], selv)

    def compute(slot):
        @plsc.parallel_loop(0, C // L, unroll=2)
        def _jloop(j):
            msk = selv[0, pl.ds(j * L, L)] > 0.5

            @plsc.parallel_loop(0, W, unroll=16)
            def _wloop(w):
                l = lbuf[slot, 0, w, pl.ds(j * L, L)]
                g = gbuf[slot, 0, w, pl.ds(j * L, L)]
                lbuf[slot, 0, w, pl.ds(j * L, L)] = jnp.where(msk, g, l)
                gbuf[slot, 0, w, pl.ds(j * L, L)] = jnp.where(msk, l, g)

    for i in range(RW):
        slot = i % SC_S
        a, b = in_cp(i, slot)
        a.wait()
        b.wait()
        compute(slot)
        a, b = out_cp(i, slot)
        a.start()
        b.start()
        pf = i + SC_PF
        if pf < RW:
            s2 = pf % SC_S
            if pf >= SC_S:
                a, b = out_cp(pf - SC_S, s2)
                a.wait()
                b.wait()
            a, b = in_cp(pf, s2)
            a.start()
            b.start()

    for k in range(SC_S):
        a, b = out_cp(RW - SC_S + k, (RW - SC_S + k) % SC_S)
        a.wait()
        b.wait()


def kernel(lst, gui, mask, W1, b1, W2, b2, Wfc, bfc):
    m, s = pl.pallas_call(
        _mlp_body,
        out_shape=(
            jax.ShapeDtypeStruct((N, C), jnp.float32),
            jax.ShapeDtypeStruct((N, C), jnp.float32),
        ),
    )(mask, W1, b1.reshape(1, C), W2, b2.reshape(1, C),
      Wfc, bfc.reshape(1, C))

    lst3 = lst.transpose(0, 2, 3, 1).reshape(N * H, W, C)
    gui3 = gui.transpose(0, 2, 3, 1).reshape(N * H, W, C)

    sc_exchange = pl.kernel(
        _sc_body,
        out_type=(
            jax.ShapeDtypeStruct((N * H, W, C), jnp.float32),
            jax.ShapeDtypeStruct((N * H, W, C), jnp.float32),
        ),
        mesh=plsc.VectorSubcoreMesh(core_axis_name="c", subcore_axis_name="s"),
        scratch_types=[
            pltpu.VMEM((SC_S, 1, W, C), jnp.float32),
            pltpu.VMEM((SC_S, 1, W, C), jnp.float32),
            pltpu.VMEM((1, C), jnp.float32),
            pltpu.SemaphoreType.DMA((SC_S,)),
            pltpu.SemaphoreType.DMA((SC_S,)),
            pltpu.SemaphoreType.DMA((SC_S,)),
            pltpu.SemaphoreType.DMA((SC_S,)),
        ],
    )
    out_l3, out_g3 = sc_exchange(s, lst3, gui3)

    out_lst = out_l3.reshape(N, H, W, C).transpose(0, 3, 1, 2)
    out_gui = out_g3.reshape(N, H, W, C).transpose(0, 3, 1, 2)
    return (out_lst, out_gui, m)


# speedup vs baseline: 3.6082x; 1.6522x over previous
import jax
import jax.numpy as jnp
from jax.experimental import pallas as pl
from jax.experimental.pallas import tpu as pltpu

N, C, H, W = 16, 512, 32, 32
BN = 2


def _fused_body(mask_ref, w1_ref, b1_ref, w2_ref, b2_ref, wfc_ref, bfc_ref,
                lst_ref, gui_ref, m_ref, out_lst_ref, out_gui_ref, sel_ref):
    n = pl.program_id(0)

    @pl.when(n == 0)
    def _mlp():
        h = jax.nn.relu(
            jnp.dot(mask_ref[...], w1_ref[...],
                    preferred_element_type=jnp.float32) + b1_ref[...])
        m = jax.nn.sigmoid(
            jnp.dot(h, w2_ref[...],
                    preferred_element_type=jnp.float32) + b2_ref[...])
        s = jax.nn.sigmoid(
            jnp.dot(m, wfc_ref[...],
                    preferred_element_type=jnp.float32) + bfc_ref[...])
        m_ref[...] = m
        sel_ref[...] = s

    rows = [sel_ref[n * BN + j, :][None, :] for j in range(BN)]
    cond = (jnp.concatenate(rows, axis=0) > 0.5)[:, None, None, :]
    l = lst_ref[...]
    g = gui_ref[...]
    out_lst_ref[...] = jnp.where(cond, g, l)
    out_gui_ref[...] = jnp.where(cond, l, g)


def kernel(lst, gui, mask, W1, b1, W2, b2, Wfc, bfc):
    lst_t = lst.transpose(0, 2, 3, 1)
    gui_t = gui.transpose(0, 2, 3, 1)

    m, out_lst_t, out_gui_t = pl.pallas_call(
        _fused_body,
        grid=(N // BN,),
        in_specs=[
            pl.BlockSpec((N, 1024), lambda n: (0, 0)),
            pl.BlockSpec((1024, C), lambda n: (0, 0)),
            pl.BlockSpec((1, C), lambda n: (0, 0)),
            pl.BlockSpec((C, C), lambda n: (0, 0)),
            pl.BlockSpec((1, C), lambda n: (0, 0)),
            pl.BlockSpec((C, C), lambda n: (0, 0)),
            pl.BlockSpec((1, C), lambda n: (0, 0)),
            pl.BlockSpec((BN, H, W, C), lambda n: (n, 0, 0, 0)),
            pl.BlockSpec((BN, H, W, C), lambda n: (n, 0, 0, 0)),
        ],
        out_specs=[
            pl.BlockSpec((N, C), lambda n: (0, 0)),
            pl.BlockSpec((BN, H, W, C), lambda n: (n, 0, 0, 0)),
            pl.BlockSpec((BN, H, W, C), lambda n: (n, 0, 0, 0)),
        ],
        out_shape=(
            jax.ShapeDtypeStruct((N, C), jnp.float32),
            jax.ShapeDtypeStruct((N, H, W, C), jnp.float32),
            jax.ShapeDtypeStruct((N, H, W, C), jnp.float32),
        ),
        scratch_shapes=[pltpu.VMEM((N, C), jnp.float32)],
    )(mask, W1, b1.reshape(1, C), W2, b2.reshape(1, C),
      Wfc, bfc.reshape(1, C), lst_t, gui_t)

    return (out_lst_t.transpose(0, 3, 1, 2),
            out_gui_t.transpose(0, 3, 1, 2), m)
